# baseline (speedup 1.0000x reference)
import functools

import jax
import jax.numpy as jnp
from jax import lax
from jax.experimental import pallas as pl
from jax.experimental.pallas import tpu as pltpu
from jax.experimental.pallas import tpu_sc as plsc

N = 10000
E = 320000
D = 128
G = 128

NC = 2
NS = 16
NW = NC * NS
CH = 64
NCH = 160
EPT = NCH * CH
EPAD = NW * EPT
NSL = 4
NSLAB = NCH // 8
NPAD = 10240
ZPT = NPAD // NS
ZCH = ZPT // CH

_sc_mesh = plsc.VectorSubcoreMesh(
    core_axis_name="c", subcore_axis_name="s", num_cores=NC, num_subcores=NS)


@functools.partial(
    pl.kernel,
    out_type=jax.ShapeDtypeStruct((NC, NPAD, D), jnp.float32),
    mesh=_sc_mesh,
    scratch_types=[
        pltpu.VMEM((2, 8, CH), jnp.int32),
        pltpu.VMEM((2, 8, CH), jnp.int32),
        pltpu.VMEM((NSL, CH, D), jnp.float32),
        pltpu.VMEM_SHARED((NPAD, D), jnp.float32),
    ] + [pltpu.SemaphoreType.DMA] * (4 + 2 * NSL),
)
def _edge_agg(h_hbm, src_hbm, dst_hbm, out_hbm, sring, dring, rows_v,
              acc_sh, *sems):
    is_sem = sems[:2]
    id_sem = sems[2:4]
    gsem = sems[4:4 + NSL]
    ssem = sems[4 + NSL:]
    cid = lax.axis_index("c")
    sid = lax.axis_index("s")
    wid = sid * NC + cid
    sbase = wid * NSLAB

    def issue_src_slab(m, k):
        pltpu.async_copy(src_hbm.at[sbase + m], sring.at[k], is_sem[k])

    def wait_src_slab(k):
        pltpu.make_async_copy(src_hbm.at[0], sring.at[k], is_sem[k]).wait()

    def issue_dst_slab(m, k):
        pltpu.async_copy(dst_hbm.at[sbase + m], dring.at[k], id_sem[k])

    def wait_dst_slab(k):
        pltpu.make_async_copy(dst_hbm.at[0], dring.at[k], id_sem[k]).wait()

    def issue_gather(k, row, j):
        pltpu.async_copy(h_hbm.at[sring.at[k, row]], rows_v.at[j], gsem[j])

    def wait_gather(j):
        pltpu.make_async_copy(h_hbm.at[sring.at[0, 0]], rows_v.at[j],
                              gsem[j]).wait()

    def issue_scatter(k, row, j):
        pltpu.async_copy(rows_v.at[j], acc_sh.at[dring.at[k, row]], ssem[j],
                         add=True)

    def wait_scatter(j):
        pltpu.make_async_copy(rows_v.at[j], acc_sh.at[dring.at[0, 0]],
                              ssem[j]).wait()

    issue_src_slab(0, 0)
    issue_dst_slab(0, 0)

    zeros16 = jnp.zeros((16,), jnp.float32)

    @pl.loop(0, CH)
    def _zero_rows(rr):
        @pl.loop(0, D // 16)
        def _zero_cols(cc):
            rows_v[0, rr, pl.ds(cc * 16, 16)] = zeros16

    @pl.loop(0, ZCH)
    def _zero_acc(z):
        pltpu.sync_copy(rows_v.at[0], acc_sh.at[pl.ds(sid * ZPT + z * CH, CH)])

    plsc.subcore_barrier()

    wait_src_slab(0)
    issue_gather(0, 0, 0)
    issue_gather(0, 1, 1)

    def slab_unit(m, k, first, last):
        kn = 1 - k
        wait_gather(0)
        wait_gather(1)
        if not last:
            issue_src_slab(m + 1, kn)
        wait_dst_slab(k)
        issue_scatter(k, 0, 0)
        issue_scatter(k, 1, 1)
        if not first:
            wait_scatter(2)
            wait_scatter(3)
        if not last:
            issue_dst_slab(m + 1, kn)
        issue_gather(k, 2, 2)
        issue_gather(k, 3, 3)
        wait_gather(2)
        wait_gather(3)
        issue_scatter(k, 2, 2)
        issue_scatter(k, 3, 3)
        wait_scatter(0)
        wait_scatter(1)
        issue_gather(k, 4, 0)
        issue_gather(k, 5, 1)
        wait_gather(0)
        wait_gather(1)
        issue_scatter(k, 4, 0)
        issue_scatter(k, 5, 1)
        wait_scatter(2)
        wait_scatter(3)
        issue_gather(k, 6, 2)
        issue_gather(k, 7, 3)
        wait_gather(2)
        wait_gather(3)
        issue_scatter(k, 6, 2)
        issue_scatter(k, 7, 3)
        wait_scatter(0)
        wait_scatter(1)
        if not last:
            wait_src_slab(kn)
            issue_gather(kn, 0, 0)
            issue_gather(kn, 1, 1)

    slab_unit(0, 0, first=True, last=False)

    @pl.loop(0, (NSLAB - 2) // 2)
    def _slabs(t):
        slab_unit(2 * t + 1, 1, first=False, last=False)
        slab_unit(2 * t + 2, 0, first=False, last=False)

    slab_unit(NSLAB - 1, 1, first=False, last=True)
    wait_scatter(2)
    wait_scatter(3)

    plsc.subcore_barrier()

    pltpu.sync_copy(acc_sh.at[pl.ds(sid * ZPT, ZPT)],
                    out_hbm.at[cid, pl.ds(sid * ZPT, ZPT)])


BN = 2000
NB = N // BN


def _layer_body(h_ref, p0_ref, p1_ref, w_ref, b_ref, o_ref):
    s = h_ref[...] + p0_ref[...] + p1_ref[...]
    y = jnp.dot(s, w_ref[...], preferred_element_type=jnp.float32) + b_ref[...]
    o_ref[...] = jnp.maximum(y, 0.0)


def _layer_tc(h, p0, p1, W, b2d):
    return pl.pallas_call(
        _layer_body,
        grid=(NB,),
        in_specs=[
            pl.BlockSpec((BN, D), lambda i: (i, 0)),
            pl.BlockSpec((BN, D), lambda i: (i, 0)),
            pl.BlockSpec((BN, D), lambda i: (i, 0)),
            pl.BlockSpec((D, D), lambda i: (0, 0)),
            pl.BlockSpec((1, D), lambda i: (0, 0)),
        ],
        out_specs=pl.BlockSpec((BN, D), lambda i: (i, 0)),
        out_shape=jax.ShapeDtypeStruct((N, D), jnp.float32),
    )(h, p0, p1, W, b2d)


def _final_body(h_ref, p0_ref, p1_ref, w2_ref, b2_ref, batch_ref, wg_ref,
                bg_ref, o_ref, sums, counts):
    i = pl.program_id(0)

    @pl.when(i == 0)
    def _():
        sums[...] = jnp.zeros_like(sums)
        counts[...] = jnp.zeros_like(counts)

    s = h_ref[...] + p0_ref[...] + p1_ref[...]
    h2 = jnp.maximum(
        jnp.dot(s, w2_ref[...], preferred_element_type=jnp.float32)
        + b2_ref[...], 0.0)

    bt = batch_ref[...].reshape(1, BN)
    gidx = lax.broadcasted_iota(jnp.int32, (G, BN), 0)
    P = (bt == gidx).astype(jnp.float32)
    sums[...] += jnp.dot(P, h2, preferred_element_type=jnp.float32)
    counts[...] += jnp.broadcast_to(jnp.sum(P, axis=1, keepdims=True), (G, D))

    @pl.when(i == NB - 1)
    def _():
        hg = sums[...] / jnp.maximum(counts[...], 1.0)
        o_ref[...] = (jnp.dot(hg, wg_ref[...], preferred_element_type=jnp.float32)
                      + bg_ref[...])


def _final_tc(h1, p0, p1, W2, b2d, batch3d, Wg, bg2d):
    return pl.pallas_call(
        _final_body,
        grid=(NB,),
        in_specs=[
            pl.BlockSpec((BN, D), lambda i: (i, 0)),
            pl.BlockSpec((BN, D), lambda i: (i, 0)),
            pl.BlockSpec((BN, D), lambda i: (i, 0)),
            pl.BlockSpec((D, D), lambda i: (0, 0)),
            pl.BlockSpec((1, D), lambda i: (0, 0)),
            pl.BlockSpec((1, 1, BN), lambda i: (i, 0, 0)),
            pl.BlockSpec((D, D), lambda i: (0, 0)),
            pl.BlockSpec((1, D), lambda i: (0, 0)),
        ],
        out_specs=pl.BlockSpec((G, D), lambda i: (0, 0)),
        out_shape=jax.ShapeDtypeStruct((G, D), jnp.float32),
        scratch_shapes=[
            pltpu.VMEM((G, D), jnp.float32),
            pltpu.VMEM((G, D), jnp.float32),
        ],
    )(h1, p0, p1, W2, b2d, batch3d, Wg, bg2d)


def kernel(x, edge_index, batch, W1, b1, W2, b2, Wg, bg):
    pad = EPAD - E
    apad = jnp.arange(pad, dtype=jnp.int32)
    src = jnp.concatenate(
        [edge_index[0].astype(jnp.int32),
         (apad * 13) % N]).reshape(EPAD // (8 * CH), 8, CH)
    dst = jnp.concatenate(
        [edge_index[1].astype(jnp.int32),
         N + apad % (NPAD - N)]).reshape(EPAD // (8 * CH), 8, CH)
    batch3d = batch.astype(jnp.int32).reshape(NB, 1, BN)

    p = _edge_agg(x, src, dst)
    h1 = _layer_tc(x, p[0, :N], p[1, :N], W1, b1.reshape(1, D))
    q = _edge_agg(h1, src, dst)
    return _final_tc(h1, q[0, :N], q[1, :N], W2, b2.reshape(1, D),
                     batch3d, Wg, bg.reshape(1, D))


# speedup vs baseline: 1.0761x; 1.0761x over previous
import functools

import jax
import jax.numpy as jnp
from jax import lax
from jax.experimental import pallas as pl
from jax.experimental.pallas import tpu as pltpu
from jax.experimental.pallas import tpu_sc as plsc

N = 10000
E = 320000
D = 128
G = 128

NC = 2
NS = 16
NW = NC * NS
CH = 64
NCHB = 156
XTRA = (E - NW * NCHB * CH) // CH
NSL = 4
NPAIR = NCHB // NSL
NPAD = 10240
ZPT = NPAD // NS
ZCH = ZPT // CH

_sc_mesh = plsc.VectorSubcoreMesh(
    core_axis_name="c", subcore_axis_name="s", num_cores=NC, num_subcores=NS)


@functools.partial(
    pl.kernel,
    out_type=jax.ShapeDtypeStruct((NC, NPAD, D), jnp.float32),
    mesh=_sc_mesh,
    scratch_types=[
        pltpu.VMEM((NSL, CH), jnp.int32),
        pltpu.VMEM((NSL, CH), jnp.int32),
        pltpu.VMEM((NSL, CH, D), jnp.float32),
        pltpu.VMEM_SHARED((NPAD, D), jnp.float32),
    ] + [pltpu.SemaphoreType.DMA] * (4 * NSL),
)
def _edge_agg(h_hbm, src_hbm, dst_hbm, out_hbm, sring, dring, rows_v,
              acc_sh, *sems):
    is_sem = sems[:NSL]
    id_sem = sems[NSL:2 * NSL]
    gsem = sems[2 * NSL:3 * NSL]
    ssem = sems[3 * NSL:]
    cid = lax.axis_index("c")
    sid = lax.axis_index("s")
    wid = sid * NC + cid
    base = wid * (NCHB * CH) + jnp.minimum(wid, XTRA) * CH

    def issue_src(c, j):
        pltpu.async_copy(src_hbm.at[pl.ds(base + c * CH, CH)], sring.at[j],
                         is_sem[j])

    def issue_dst(c, j):
        pltpu.async_copy(dst_hbm.at[pl.ds(base + c * CH, CH)], dring.at[j],
                         id_sem[j])

    def wait_src(j):
        pltpu.make_async_copy(src_hbm.at[pl.ds(0, CH)], sring.at[j],
                              is_sem[j]).wait()

    def wait_dst(j):
        pltpu.make_async_copy(dst_hbm.at[pl.ds(0, CH)], dring.at[j],
                              id_sem[j]).wait()

    def issue_gather(j):
        pltpu.async_copy(h_hbm.at[sring.at[j]], rows_v.at[j], gsem[j])

    def wait_gather(j):
        pltpu.make_async_copy(h_hbm.at[sring.at[0]], rows_v.at[j],
                              gsem[j]).wait()

    def issue_scatter(j):
        pltpu.async_copy(rows_v.at[j], acc_sh.at[dring.at[j]], ssem[j],
                         add=True)

    def wait_scatter(j):
        pltpu.make_async_copy(rows_v.at[j], acc_sh.at[dring.at[j]],
                              ssem[j]).wait()

    for j in range(NSL):
        issue_src(j, j)
    for j in (0, 1):
        issue_dst(j, j)

    zeros16 = jnp.zeros((16,), jnp.float32)

    @pl.loop(0, CH)
    def _zero_rows(rr):
        @pl.loop(0, D // 16)
        def _zero_cols(cc):
            rows_v[0, rr, pl.ds(cc * 16, 16)] = zeros16

    @pl.loop(0, ZCH)
    def _zero_acc(z):
        pltpu.sync_copy(rows_v.at[0], acc_sh.at[pl.ds(sid * ZPT + z * CH, CH)])

    plsc.subcore_barrier()

    for j in (0, 1):
        wait_src(j)
        issue_gather(j)

    for j in (0, 1):
        wait_gather(j)
        issue_src(4 + j, j)
        wait_dst(j)
        issue_scatter(j)
    for j in (2, 3):
        issue_dst(j, j)
        wait_src(j)
        issue_gather(j)
    for j in (2, 3):
        wait_gather(j)
        issue_src(4 + j, j)
        wait_dst(j)
        issue_scatter(j)
    for j in (0, 1):
        wait_scatter(j)
        issue_dst(4 + j, j)
        wait_src(j)
        issue_gather(j)

    @pl.loop(1, NPAIR - 1)
    def _pairs(p):
        c0 = p * NSL
        for j in (0, 1):
            wait_gather(j)
            issue_src(c0 + 4 + j, j)
            wait_dst(j)
            issue_scatter(j)
        for j in (2, 3):
            wait_scatter(j)
            issue_dst(c0 + j, j)
            wait_src(j)
            issue_gather(j)
        for j in (2, 3):
            wait_gather(j)
            issue_src(c0 + 4 + j, j)
            wait_dst(j)
            issue_scatter(j)
        for j in (0, 1):
            wait_scatter(j)
            issue_dst(c0 + 4 + j, j)
            wait_src(j)
            issue_gather(j)

    c0 = NCHB - NSL
    for j in (0, 1):
        wait_gather(j)
        wait_dst(j)
        issue_scatter(j)
    for j in (2, 3):
        wait_scatter(j)
        issue_dst(c0 + j, j)
        wait_src(j)
        issue_gather(j)
    for j in (2, 3):
        wait_gather(j)
        wait_dst(j)
        issue_scatter(j)
    for j in range(NSL):
        wait_scatter(j)

    @pl.when(wid < XTRA)
    def _extra():
        issue_src(NCHB, 0)
        issue_dst(NCHB, 0)
        wait_src(0)
        issue_gather(0)
        wait_gather(0)
        wait_dst(0)
        issue_scatter(0)
        wait_scatter(0)

    plsc.subcore_barrier()

    pltpu.sync_copy(acc_sh.at[pl.ds(sid * ZPT, ZPT)],
                    out_hbm.at[cid, pl.ds(sid * ZPT, ZPT)])


BN = 2000
NB = N // BN


def _layer_body(h_ref, p0_ref, p1_ref, w_ref, b_ref, o_ref):
    s = h_ref[...] + p0_ref[0] + p1_ref[0]
    y = jnp.dot(s, w_ref[...], preferred_element_type=jnp.float32) + b_ref[...]
    o_ref[...] = jnp.maximum(y, 0.0)


def _layer_tc(h, p, W, b2d):
    return pl.pallas_call(
        _layer_body,
        grid=(NB,),
        in_specs=[
            pl.BlockSpec((BN, D), lambda i: (i, 0)),
            pl.BlockSpec((1, BN, D), lambda i: (0, i, 0)),
            pl.BlockSpec((1, BN, D), lambda i: (1, i, 0)),
            pl.BlockSpec((D, D), lambda i: (0, 0)),
            pl.BlockSpec((1, D), lambda i: (0, 0)),
        ],
        out_specs=pl.BlockSpec((BN, D), lambda i: (i, 0)),
        out_shape=jax.ShapeDtypeStruct((N, D), jnp.float32),
    )(h, p, p, W, b2d)


def _final_body(h_ref, p0_ref, p1_ref, w2_ref, b2_ref, batch_ref, wg_ref,
                bg_ref, o_ref, sums, counts):
    i = pl.program_id(0)

    @pl.when(i == 0)
    def _():
        sums[...] = jnp.zeros_like(sums)
        counts[...] = jnp.zeros_like(counts)

    s = h_ref[...] + p0_ref[0] + p1_ref[0]
    h2 = jnp.maximum(
        jnp.dot(s, w2_ref[...], preferred_element_type=jnp.float32)
        + b2_ref[...], 0.0)

    bt = batch_ref[...].reshape(1, BN)
    gidx = lax.broadcasted_iota(jnp.int32, (G, BN), 0)
    P = (bt == gidx).astype(jnp.float32)
    sums[...] += jnp.dot(P, h2, preferred_element_type=jnp.float32)
    counts[...] += jnp.broadcast_to(jnp.sum(P, axis=1, keepdims=True), (G, D))

    @pl.when(i == NB - 1)
    def _():
        hg = sums[...] / jnp.maximum(counts[...], 1.0)
        o_ref[...] = (jnp.dot(hg, wg_ref[...], preferred_element_type=jnp.float32)
                      + bg_ref[...])


def _final_tc(h1, q, W2, b2d, batch3d, Wg, bg2d):
    return pl.pallas_call(
        _final_body,
        grid=(NB,),
        in_specs=[
            pl.BlockSpec((BN, D), lambda i: (i, 0)),
            pl.BlockSpec((1, BN, D), lambda i: (0, i, 0)),
            pl.BlockSpec((1, BN, D), lambda i: (1, i, 0)),
            pl.BlockSpec((D, D), lambda i: (0, 0)),
            pl.BlockSpec((1, D), lambda i: (0, 0)),
            pl.BlockSpec((1, 1, BN), lambda i: (i, 0, 0)),
            pl.BlockSpec((D, D), lambda i: (0, 0)),
            pl.BlockSpec((1, D), lambda i: (0, 0)),
        ],
        out_specs=pl.BlockSpec((G, D), lambda i: (0, 0)),
        out_shape=jax.ShapeDtypeStruct((G, D), jnp.float32),
        scratch_shapes=[
            pltpu.VMEM((G, D), jnp.float32),
            pltpu.VMEM((G, D), jnp.float32),
        ],
    )(h1, q, q, W2, b2d, batch3d, Wg, bg2d)


def kernel(x, edge_index, batch, W1, b1, W2, b2, Wg, bg):
    src = edge_index[0].astype(jnp.int32)
    dst = edge_index[1].astype(jnp.int32)
    batch3d = batch.astype(jnp.int32).reshape(NB, 1, BN)

    p = _edge_agg(x, src, dst)
    h1 = _layer_tc(x, p, W1, b1.reshape(1, D))
    q = _edge_agg(h1, src, dst)
    return _final_tc(h1, q, W2, b2.reshape(1, D),
                     batch3d, Wg, bg.reshape(1, D))
